# Initial kernel scaffold; baseline (speedup 1.0000x reference)
import jax
import jax.numpy as jnp
from jax.experimental import pallas as pl


def kernel(x, Wr, br, W1, b1, W2, b2):
    raise NotImplementedError("write your pallas kernel here")


# speedup vs baseline: 3.3110x; 3.3110x over previous
import functools

import jax
import jax.numpy as jnp
from jax.experimental import pallas as pl
from jax.experimental.pallas import tpu as pltpu

DIM = 768
HIDDEN = 4 * DIM
NUM_EXPERTS = 8
TOPK = 8
B_TOTAL = 4096
SEL = NUM_EXPERTS * TOPK

TOKEN_BLOCK = 512
HID_BLOCK = 512
N_HID_BLOCKS = HIDDEN // HID_BLOCK

_INTERPRET = False


def _router_body(x_ref, wr_ref, br_ref, out_ref):
    logits = jax.lax.dot_general(
        wr_ref[...], x_ref[...], (((1,), (1,)), ((), ())),
        preferred_element_type=jnp.float32)
    logits = logits + br_ref[...]
    m = jnp.max(logits, axis=0, keepdims=True)
    e = jnp.exp(logits - m)
    s = jnp.sum(e, axis=0, keepdims=True)
    out_ref[...] = e / s


def _topk_body(scores_ref, idx_ref, val_ref):
    s = scores_ref[...]
    lane = jax.lax.broadcasted_iota(jnp.int32, (NUM_EXPERTS, B_TOTAL), 1)
    lane_out = jax.lax.broadcasted_iota(jnp.int32, (NUM_EXPERTS, 128), 1)
    idx_acc = jnp.zeros((NUM_EXPERTS, 128), jnp.int32)
    val_acc = jnp.zeros((NUM_EXPERTS, 128), jnp.float32)
    for k in range(TOPK):
        m = jnp.max(s, axis=1, keepdims=True)
        cand = jnp.where(s == m, lane, B_TOTAL)
        amin = jnp.min(cand, axis=1, keepdims=True)
        idx_acc = jnp.where(lane_out == k, jnp.broadcast_to(amin, idx_acc.shape),
                            idx_acc)
        val_acc = jnp.where(lane_out == k, jnp.broadcast_to(m, val_acc.shape),
                            val_acc)
        s = jnp.where(lane == amin, -jnp.inf, s)
    idx_ref[...] = idx_acc
    val_ref[...] = val_acc


def _gather_body(idx_ref, x_ref, out_ref):
    del idx_ref
    out_ref[...] = x_ref[...]


def _ffn_body(xg_ref, w1_ref, b1_ref, w2_ref, b2_ref, sc_ref, y_ref, acc_ref):
    hb = pl.program_id(1)

    @pl.when(hb == 0)
    def _():
        acc_ref[...] = jnp.zeros_like(acc_ref)

    h = jax.lax.dot_general(
        xg_ref[...], w1_ref[0], (((1,), (1,)), ((), ())),
        preferred_element_type=jnp.float32)
    h = jnp.maximum(h + b1_ref[0], 0.0)
    acc_ref[...] += jax.lax.dot_general(
        h, w2_ref[0], (((1,), (1,)), ((), ())),
        preferred_element_type=jnp.float32)

    @pl.when(hb == N_HID_BLOCKS - 1)
    def _():
        y_ref[...] = (acc_ref[...] + b2_ref[0]) * sc_ref[...]


def _scatter_body(idx_ref, y_ref, out_ref):
    out_ref[...] = jnp.zeros_like(out_ref)

    def body(i, carry):
        t = idx_ref[i]
        out_ref[pl.ds(t, 1), :] += y_ref[pl.ds(i, 1), :]
        return carry

    jax.lax.fori_loop(0, SEL, body, 0)


def kernel(x, Wr, br, W1, b1, W2, b2):
    B, D = x.shape
    E = Wr.shape[0]

    scores_t = pl.pallas_call(
        _router_body,
        grid=(B // TOKEN_BLOCK,),
        in_specs=[
            pl.BlockSpec((TOKEN_BLOCK, D), lambda i: (i, 0)),
            pl.BlockSpec((E, D), lambda i: (0, 0)),
            pl.BlockSpec((E, 1), lambda i: (0, 0)),
        ],
        out_specs=pl.BlockSpec((E, TOKEN_BLOCK), lambda i: (0, i)),
        out_shape=jax.ShapeDtypeStruct((E, B), jnp.float32),
        interpret=_INTERPRET,
    )(x, Wr, br.reshape(E, 1))

    idx_pad, val_pad = pl.pallas_call(
        _topk_body,
        out_shape=(jax.ShapeDtypeStruct((E, 128), jnp.int32),
                   jax.ShapeDtypeStruct((E, 128), jnp.float32)),
        interpret=_INTERPRET,
    )(scores_t)
    idx_flat = idx_pad[:, :TOPK].reshape(SEL)
    scores_b = val_pad[:, :TOPK].reshape(SEL, 1)

    xg = pl.pallas_call(
        _gather_body,
        grid_spec=pltpu.PrefetchScalarGridSpec(
            num_scalar_prefetch=1,
            grid=(SEL,),
            in_specs=[pl.BlockSpec((1, 1, D), lambda i, idx: (idx[i], 0, 0))],
            out_specs=pl.BlockSpec((1, 1, D), lambda i, idx: (i, 0, 0)),
        ),
        out_shape=jax.ShapeDtypeStruct((SEL, 1, D), jnp.float32),
        interpret=_INTERPRET,
    )(idx_flat, x.reshape(B, 1, D)).reshape(SEL, D)

    sc_bcast = jnp.broadcast_to(scores_b, (SEL, D))
    y = pl.pallas_call(
        _ffn_body,
        grid=(E, N_HID_BLOCKS),
        in_specs=[
            pl.BlockSpec((TOPK, D), lambda e, h: (e, 0)),
            pl.BlockSpec((1, HID_BLOCK, D), lambda e, h: (e, h, 0)),
            pl.BlockSpec((1, 1, HID_BLOCK), lambda e, h: (e, 0, h)),
            pl.BlockSpec((1, D, HID_BLOCK), lambda e, h: (e, 0, h)),
            pl.BlockSpec((1, 1, D), lambda e, h: (e, 0, 0)),
            pl.BlockSpec((TOPK, D), lambda e, h: (e, 0)),
        ],
        out_specs=pl.BlockSpec((TOPK, D), lambda e, h: (e, 0)),
        out_shape=jax.ShapeDtypeStruct((SEL, D), jnp.float32),
        scratch_shapes=[pltpu.VMEM((TOPK, D), jnp.float32)],
        interpret=_INTERPRET,
    )(xg, W1, b1.reshape(E, 1, HIDDEN), W2, b2.reshape(E, 1, D), sc_bcast)

    out = pl.pallas_call(
        _scatter_body,
        grid_spec=pltpu.PrefetchScalarGridSpec(
            num_scalar_prefetch=1,
            grid=(1,),
            in_specs=[pl.BlockSpec((SEL, D), lambda i, idx: (0, 0))],
            out_specs=pl.BlockSpec((B, D), lambda i, idx: (0, 0)),
        ),
        out_shape=jax.ShapeDtypeStruct((B, D), jnp.float32),
        interpret=_INTERPRET,
    )(idx_flat, y)
    return out
